# Initial kernel scaffold; baseline (speedup 1.0000x reference)
import jax
import jax.numpy as jnp
from jax.experimental import pallas as pl


def kernel(tokens, att_mask, W, P, gamma, beta):
    raise NotImplementedError("write your pallas kernel here")


# speedup vs baseline: 1.1929x; 1.1929x over previous
import functools

import jax
import jax.numpy as jnp
from jax import lax
from jax.experimental import pallas as pl
from jax.experimental.pallas import tpu as pltpu
from jax.experimental.pallas import tpu_sc as plsc

_NC = 2
_NS = 16
_NW = _NC * _NS


def _sc_gather(tokens_flat, W):
    N = tokens_flat.shape[0]
    V, D = W.shape
    per_w = N // _NW
    CH = 64
    n_ch = per_w // CH

    mesh = plsc.VectorSubcoreMesh(core_axis_name="c", subcore_axis_name="s")

    @functools.partial(
        pl.kernel,
        mesh=mesh,
        out_type=jax.ShapeDtypeStruct((N, D), jnp.float32),
        scratch_types=[
            pltpu.VMEM((per_w,), jnp.int32),
            pltpu.VMEM((CH, D), jnp.float32),
            pltpu.SemaphoreType.DMA,
        ],
    )
    def k(tokens_hbm, W_hbm, out_hbm, idx_v, rows_v, sem):
        wid = lax.axis_index("s") * _NC + lax.axis_index("c")
        base = wid * per_w
        pltpu.sync_copy(tokens_hbm.at[pl.ds(base, per_w)], idx_v)
        for c in range(n_ch):
            pltpu.async_copy(
                W_hbm.at[idx_v.at[pl.ds(c * CH, CH)]], rows_v, sem
            ).wait()
            pltpu.sync_copy(rows_v, out_hbm.at[pl.ds(base + c * CH, CH)])

    return k(tokens_flat, W)


def _tc_ln(g, P, gamma, beta, d_model):
    N, D = g.shape
    S = P.shape[0]
    BLK = 256
    scale = float(d_model) ** 0.5

    def body(g_ref, p_ref, gm_ref, bt_ref, o_ref):
        x = g_ref[...] * scale + p_ref[...]
        mu = jnp.mean(x, axis=-1, keepdims=True)
        var = jnp.mean((x - mu) ** 2, axis=-1, keepdims=True)
        xn = (x - mu) / jnp.sqrt(var + 1e-12)
        o_ref[...] = xn * gm_ref[...] + bt_ref[...]

    return pl.pallas_call(
        body,
        grid=(N // BLK,),
        in_specs=[
            pl.BlockSpec((BLK, D), lambda i: (i, 0)),
            pl.BlockSpec((BLK, D), lambda i: (i % (S // BLK), 0)),
            pl.BlockSpec((1, D), lambda i: (0, 0)),
            pl.BlockSpec((1, D), lambda i: (0, 0)),
        ],
        out_specs=pl.BlockSpec((BLK, D), lambda i: (i, 0)),
        out_shape=jax.ShapeDtypeStruct((N, D), jnp.float32),
    )(g, P, gamma.reshape(1, D), beta.reshape(1, D))


def kernel(tokens, att_mask, W, P, gamma, beta):
    B, S = tokens.shape
    D = W.shape[1]
    g = _sc_gather(tokens.reshape(-1), W)
    out = _tc_ln(g, P, gamma, beta, D)
    return out.reshape(B, S, D), att_mask


# speedup vs baseline: 1.2021x; 1.0077x over previous
import functools

import jax
import jax.numpy as jnp
from jax import lax
from jax.experimental import pallas as pl
from jax.experimental.pallas import tpu as pltpu
from jax.experimental.pallas import tpu_sc as plsc

_NC = 2
_NS = 16
_NW = _NC * _NS


def _sc_gather(tokens_flat, W):
    N = tokens_flat.shape[0]
    V, D = W.shape
    per_w = N // _NW
    CH = 32
    n_ch = per_w // CH

    mesh = plsc.VectorSubcoreMesh(core_axis_name="c", subcore_axis_name="s")

    @functools.partial(
        pl.kernel,
        mesh=mesh,
        out_type=jax.ShapeDtypeStruct((N, D), jnp.float32),
        scratch_types=[
            pltpu.VMEM((per_w,), jnp.int32),
            pltpu.VMEM((CH, D), jnp.float32),
            pltpu.VMEM((CH, D), jnp.float32),
            pltpu.SemaphoreType.DMA,
            pltpu.SemaphoreType.DMA,
            pltpu.SemaphoreType.DMA,
            pltpu.SemaphoreType.DMA,
        ],
    )
    def k(tokens_hbm, W_hbm, out_hbm, idx_v, buf0, buf1, g0, g1, s0, s1):
        wid = lax.axis_index("s") * _NC + lax.axis_index("c")
        base = wid * per_w
        bufs = (buf0, buf1)
        gsems = (g0, g1)
        ssems = (s0, s1)
        pltpu.sync_copy(tokens_hbm.at[pl.ds(base, per_w)], idx_v)

        def gather(c):
            return pltpu.async_copy(
                W_hbm.at[idx_v.at[pl.ds(c * CH, CH)]], bufs[c % 2], gsems[c % 2]
            )

        def scatter(c):
            return pltpu.async_copy(
                bufs[c % 2], out_hbm.at[pl.ds(base + c * CH, CH)], ssems[c % 2]
            )

        pend_g = {0: gather(0)}
        pend_s = {}
        for c in range(n_ch):
            pend_g.pop(c).wait()
            if c + 1 < n_ch:
                if c - 1 in pend_s:
                    pend_s.pop(c - 1).wait()
                pend_g[c + 1] = gather(c + 1)
            pend_s[c] = scatter(c)
        pend_s.pop(n_ch - 1).wait()

    return k(tokens_flat, W)


def _tc_ln(g, P, gamma, beta, d_model):
    N, D = g.shape
    S = P.shape[0]
    B = N // S
    BLK = 256
    scale = float(d_model) ** 0.5

    def body(g_ref, p_ref, gm_ref, bt_ref, o_ref):
        x = g_ref[...] * scale + p_ref[...]
        mu = jnp.mean(x, axis=-1, keepdims=True)
        var = jnp.mean((x - mu) ** 2, axis=-1, keepdims=True)
        xn = (x - mu) / jnp.sqrt(var + 1e-12)
        o_ref[...] = xn * gm_ref[...] + bt_ref[...]

    n_s = S // BLK
    return pl.pallas_call(
        body,
        grid=(n_s, B),
        in_specs=[
            pl.BlockSpec((BLK, D), lambda i, b: (b * n_s + i, 0)),
            pl.BlockSpec((BLK, D), lambda i, b: (i, 0)),
            pl.BlockSpec((1, D), lambda i, b: (0, 0)),
            pl.BlockSpec((1, D), lambda i, b: (0, 0)),
        ],
        out_specs=pl.BlockSpec((BLK, D), lambda i, b: (b * n_s + i, 0)),
        out_shape=jax.ShapeDtypeStruct((N, D), jnp.float32),
    )(g, P, gamma.reshape(1, D), beta.reshape(1, D))


def kernel(tokens, att_mask, W, P, gamma, beta):
    B, S = tokens.shape
    D = W.shape[1]
    g = _sc_gather(tokens.reshape(-1), W)
    out = _tc_ln(g, P, gamma, beta, D)
    return out.reshape(B, S, D), att_mask
